# baseline (speedup 1.0000x reference)
import functools

import jax
import jax.numpy as jnp
from jax import lax
from jax.experimental import pallas as pl
from jax.experimental.pallas import tpu as pltpu
from jax.experimental.pallas import tpu_sc as plsc

_OH, _OW = 7, 7
_SCALE = 0.0625
_RSPAN = 8
_CSPAN = 11
_NTBL = _RSPAN * _CSPAN

_NP = 1024
_NSC = 32
_RPT = _NP // _NSC
_NCELL = _OH * _OW
_GRP = 4
_NROW = _GRP * 16
_OSZ = 256 * _NCELL

_INTERPRET = False


def _rowtab_body(f_ref, r_ref, *, H, W, C):
    a = f_ref[0]
    r_ref[0, 0] = a
    cur = a
    for r in range(2, _RSPAN + 1):
        d = r - 1
        shifted = jnp.concatenate(
            [a[d:], jnp.broadcast_to(a[H - 1:], (d, W, C))], axis=0)
        cur = jnp.maximum(cur, shifted)
        r_ref[r - 1, 0] = cur


def _coltab_body(r_ref, t_ref, *, H, W, C):
    s = r_ref[0, 0]
    t_ref[0, 0, 0] = s
    cur = s
    for c in range(2, _CSPAN + 1):
        d = c - 1
        shifted = jnp.concatenate(
            [s[:, d:], jnp.broadcast_to(s[:, W - 1:], (H, d, C))], axis=1)
        cur = jnp.maximum(cur, shifted)
        t_ref[0, c - 1, 0] = cur


def _build_tables(feats):
    B, H, W, C = feats.shape
    cc = 128
    rows = pl.pallas_call(
        functools.partial(_rowtab_body, H=H, W=W, C=cc),
        grid=(B, C // cc),
        in_specs=[pl.BlockSpec((1, H, W, cc), lambda b, c: (b, 0, 0, c))],
        out_specs=pl.BlockSpec((_RSPAN, 1, H, W, cc),
                               lambda b, c: (0, b, 0, 0, c)),
        out_shape=jax.ShapeDtypeStruct((_RSPAN, B, H, W, C), jnp.float32),
        interpret=_INTERPRET,
    )(feats)
    return pl.pallas_call(
        functools.partial(_coltab_body, H=H, W=W, C=cc),
        grid=(_RSPAN, B, C // cc),
        in_specs=[pl.BlockSpec((1, 1, H, W, cc),
                               lambda r, b, c: (r, b, 0, 0, c))],
        out_specs=pl.BlockSpec((1, _CSPAN, 1, H, W, cc),
                               lambda r, b, c: (r, 0, b, 0, 0, c)),
        out_shape=jax.ShapeDtypeStruct((_RSPAN, _CSPAN, B, H, W, C),
                                       jnp.float32),
        interpret=_INTERPRET,
    )(rows)


def _make_sc_kernel(C, W, BHW):
    mesh = plsc.VectorSubcoreMesh(core_axis_name="c", subcore_axis_name="s")

    @functools.partial(
        pl.kernel,
        mesh=mesh,
        compiler_params=pltpu.CompilerParams(needs_layout_passes=False),
        out_type=jax.ShapeDtypeStruct((_NP * _OSZ,), jnp.float32),
        scratch_types=[
            pltpu.VMEM((_RPT * 16,), jnp.int32),
            pltpu.VMEM((2 * _GRP * 16,), jnp.int32),
            pltpu.VMEM((_NROW,), jnp.int32),
            pltpu.VMEM((_NROW,), jnp.int32),
            pltpu.VMEM((_NROW, C), jnp.float32),
            pltpu.VMEM((_NROW, C), jnp.float32),
            pltpu.VMEM((_OSZ,), jnp.float32),
            pltpu.SemaphoreType.DMA,
            pltpu.SemaphoreType.DMA,
        ],
    )
    def sc_kernel(table_hbm, params_hbm, consts_hbm, out_hbm,
                  pv, cv, idx0, idx1, dat0, dat1, outv, sem0, sem1):
        wid = lax.axis_index("s") * 2 + lax.axis_index("c")
        r0 = wid * _RPT
        pltpu.sync_copy(params_hbm.at[pl.ds(r0 * 16, _RPT * 16)], pv)
        pltpu.sync_copy(consts_hbm, cv)

        lane = lax.iota(jnp.int32, 16)
        lane49 = lane * _NCELL

        def fd7(a):
            return (a * 9363) >> 16

        def gen(r, idxbuf):
            pbase = r * 16

            def bcast(i):
                return plsc.load_gather(pv, [lane * 0 + (pbase + i)])

            basev = bcast(0)
            y1v = bcast(1)
            hv = bcast(2)
            x1v = bcast(3)
            wv = bcast(4)
            for g in range(_GRP):
                jv = cv[pl.ds(g * 16, 16)]
                kv = cv[pl.ds(_GRP * 16 + g * 16, 16)]
                rs = y1v + fd7(jv * hv)
                re = y1v + fd7((jv + 1) * hv + (_OH - 1))
                rlen = re - rs
                cs = x1v + fd7(kv * wv)
                ce = x1v + fd7((kv + 1) * wv + (_OW - 1))
                clen = ce - cs
                tid = (rlen - 1) * _CSPAN + (clen - 1)
                flat = tid * BHW + basev + rs * W + cs
                idxbuf[pl.ds(g * 16, 16)] = flat

        def fire(idxbuf, datbuf, sem):
            return pltpu.async_copy(table_hbm.at[idxbuf], datbuf, sem)

        def reduce_out(r, datbuf):
            def cell_body(c, _):
                for ch in range(C // 16):
                    v = datbuf[c, pl.ds(ch * 16, 16)]
                    tgt = lane49 + (ch * 16 * _NCELL + c)
                    plsc.store_scatter(outv, [tgt], v)
                return 0
            lax.fori_loop(0, _NCELL, cell_body, 0)
            pltpu.sync_copy(
                outv, out_hbm.at[pl.ds((r0 + r) * _OSZ, _OSZ)])

        def pair_body(i, _):
            ra = 2 * i
            rb = ra + 1
            gen(ra, idx0)
            a = fire(idx0, dat0, sem0)
            gen(rb, idx1)
            b = fire(idx1, dat1, sem1)
            a.wait()
            reduce_out(ra, dat0)
            b.wait()
            reduce_out(rb, dat1)
            return 0

        lax.fori_loop(0, _RPT // 2, pair_body, 0)

    return sc_kernel


def kernel(features, rois):
    B, C, H, W = features.shape
    N = rois.shape[0]

    feats = jnp.transpose(features, (0, 2, 3, 1))
    table = _build_tables(feats)
    table2d = table.reshape(_NTBL * B * H * W, C)

    bidx = rois[:, 0].astype(jnp.int32)
    coords = (rois[:, 1:5] * _SCALE).astype(jnp.int32)
    x1 = jnp.clip(coords[:, 0], 0, W - 1)
    y1 = jnp.clip(coords[:, 1], 0, H - 1)
    x2 = jnp.clip(coords[:, 2], 0, W - 1)
    y2 = jnp.clip(coords[:, 3], 0, H - 1)
    h = y2 - y1 + 1
    w = x2 - x1 + 1
    base = bidx * (H * W)

    cols = jnp.stack(
        [base, y1, h, x1, w] + [jnp.zeros_like(base)] * 11, axis=1)
    pad = jnp.tile(jnp.array([[0, 0, 1, 0, 1] + [0] * 11], jnp.int32),
                   (_NP - N, 1))
    params = jnp.concatenate([cols, pad], axis=0).reshape(_NP * 16)

    cell = jnp.minimum(jnp.arange(_GRP * 16, dtype=jnp.int32), _NCELL - 1)
    consts = jnp.concatenate([cell // _OW, cell % _OW])

    sc = _make_sc_kernel(C, W, B * H * W)
    out_flat = sc(table2d, params, consts)

    out = out_flat.reshape(_NP, C, _NCELL)[:N]
    return out.reshape(N, C, _OH, _OW)
